# Initial kernel scaffold; baseline (speedup 1.0000x reference)
import jax
import jax.numpy as jnp
from jax.experimental import pallas as pl


def kernel(x, router_w, W_gate, W_up, W_down, Ws_gate, Ws_up, Ws_down):
    raise NotImplementedError("write your pallas kernel here")


# speedup vs baseline: 1.9651x; 1.9651x over previous
import functools

import jax
import jax.numpy as jnp
from jax.experimental import pallas as pl
from jax.experimental.pallas import tpu as pltpu

T = 2048
D = 768
F = 1280
E = 8
M = 512


def _combine_weights(x, router_w):
    logits = jnp.dot(x, router_w, preferred_element_type=jnp.float32)
    aff = jax.nn.sigmoid(logits)
    ii = jax.lax.broadcasted_iota(jnp.int32, aff.shape, 1)
    m1 = jnp.max(aff, axis=1, keepdims=True)
    e1 = jnp.min(jnp.where(aff == m1, ii, E), axis=1, keepdims=True)
    mask1 = ii == e1
    aff2 = jnp.where(mask1, -1.0, aff)
    m2 = jnp.max(aff2, axis=1, keepdims=True)
    e2 = jnp.min(jnp.where(aff2 == m2, ii, E), axis=1, keepdims=True)
    mask2 = ii == e2
    denom = m1 + m2
    comb = (jnp.where(mask1, m1, 0.0) + jnp.where(mask2, m2, 0.0)) / denom
    return comb


def _routed_body(x_ref, rw_ref, wg_ref, wu_ref, wd_ref, out_ref):
    e = pl.program_id(1)
    x = x_ref[...]
    comb = _combine_weights(x, rw_ref[...])
    ii = jax.lax.broadcasted_iota(jnp.int32, comb.shape, 1)
    c = jnp.sum(jnp.where(ii == e, comb, 0.0), axis=1, keepdims=True)
    g = jax.nn.silu(jnp.dot(x, wg_ref[0], preferred_element_type=jnp.float32))
    u = jnp.dot(x, wu_ref[0], preferred_element_type=jnp.float32)
    y = jnp.dot(g * u, wd_ref[0], preferred_element_type=jnp.float32)

    @pl.when(e == 0)
    def _init():
        out_ref[...] = c * y

    @pl.when(e != 0)
    def _acc():
        out_ref[...] += c * y


def _shared_body(x_ref, routed_ref, wg_ref, wu_ref, wd_ref, out_ref):
    x = x_ref[...]
    g = jax.nn.silu(jnp.dot(x, wg_ref[...], preferred_element_type=jnp.float32))
    u = jnp.dot(x, wu_ref[...], preferred_element_type=jnp.float32)
    y = jnp.dot(g * u, wd_ref[...], preferred_element_type=jnp.float32)
    out_ref[...] = routed_ref[...] + y


@jax.jit
def kernel(x, router_w, W_gate, W_up, W_down, Ws_gate, Ws_up, Ws_down):
    nt = T // M
    routed = pl.pallas_call(
        _routed_body,
        grid=(nt, E),
        in_specs=[
            pl.BlockSpec((M, D), lambda i, j: (i, 0)),
            pl.BlockSpec((D, E), lambda i, j: (0, 0)),
            pl.BlockSpec((1, D, F), lambda i, j: (j, 0, 0)),
            pl.BlockSpec((1, D, F), lambda i, j: (j, 0, 0)),
            pl.BlockSpec((1, F, D), lambda i, j: (j, 0, 0)),
        ],
        out_specs=pl.BlockSpec((M, D), lambda i, j: (i, 0)),
        out_shape=jax.ShapeDtypeStruct((T, D), jnp.float32),
        compiler_params=pltpu.CompilerParams(
            dimension_semantics=("parallel", "arbitrary"),
        ),
    )(x, router_w, W_gate, W_up, W_down)

    out = pl.pallas_call(
        _shared_body,
        grid=(nt,),
        in_specs=[
            pl.BlockSpec((M, D), lambda i: (i, 0)),
            pl.BlockSpec((M, D), lambda i: (i, 0)),
            pl.BlockSpec((D, F), lambda i: (0, 0)),
            pl.BlockSpec((D, F), lambda i: (0, 0)),
            pl.BlockSpec((F, D), lambda i: (0, 0)),
        ],
        out_specs=pl.BlockSpec((M, D), lambda i: (i, 0)),
        out_shape=jax.ShapeDtypeStruct((T, D), jnp.float32),
        compiler_params=pltpu.CompilerParams(
            dimension_semantics=("parallel",),
        ),
    )(x, routed, Ws_gate, Ws_up, Ws_down)
    return out


# speedup vs baseline: 1.9869x; 1.0111x over previous
import functools

import numpy as np
import jax
import jax.numpy as jnp
from jax import lax
from jax.experimental import pallas as pl
from jax.experimental.pallas import tpu as pltpu
from jax.experimental.pallas import tpu_sc as plsc

T = 2048
D = 768
F = 1280
E = 8
K = 2
A = T * K
MG = 128
RBUF = A + E * MG
NT = RBUF // MG
NW = 32
CHUNK = A // NW
MF = 512

_P, _Q = 32, 128
_C = _P * E
_cc = np.arange(_C)
_REP = (np.arange(_P)[:, None] == (_cc // E)[None, :]).astype(np.float32)
_EIDX = (_cc % E).astype(np.float32)[None, :]
_R256 = (((_cc % E)[:, None] == (_cc % E)[None, :])
         & ((_cc // E)[:, None] < (_cc // E)[None, :])).astype(np.float32)
_RSUM = ((_cc % E)[:, None] == np.arange(E)[None, :]).astype(np.float32)
_L128 = (np.arange(_Q)[None, :] < np.arange(_Q)[:, None]).astype(np.float32)
_L8 = (np.arange(E)[:, None] < np.arange(E)[None, :]).astype(np.float32)
_TILE8 = (np.arange(E)[:, None] == (_cc % E)[None, :]).astype(np.float32)
_REPT = ((_cc // E)[:, None] == np.arange(_P)[None, :]).astype(np.float32)
_TOK = (np.arange(A) % T).astype(np.int32)


def _router_body(x_ref, rw_ref, e1_ref, e2_ref, w1_ref, w2_ref):
    x = x_ref[...]
    logits = jnp.dot(x, rw_ref[...], preferred_element_type=jnp.float32)
    aff = jax.nn.sigmoid(logits)
    ii = jax.lax.broadcasted_iota(jnp.int32, aff.shape, 1)
    m1 = jnp.max(aff, axis=1, keepdims=True)
    e1 = jnp.min(jnp.where(aff == m1, ii, E), axis=1, keepdims=True)
    aff2 = jnp.where(ii == e1, -1.0, aff)
    m2 = jnp.max(aff2, axis=1, keepdims=True)
    e2 = jnp.min(jnp.where(aff2 == m2, ii, E), axis=1, keepdims=True)
    denom = m1 + m2
    e1_ref[...] = e1
    e2_ref[...] = e2
    w1_ref[...] = m1 / denom
    w2_ref[...] = m2 / denom


def _index_body(e_ref, rep_ref, eidx_ref, r_ref, rsum_ref, l128_ref,
                l8_ref, tile8_ref, rept_ref, pos_ref, cnt_ref):
    hi = jax.lax.Precision.HIGHEST
    dot = functools.partial(jnp.dot, precision=hi,
                            preferred_element_type=jnp.float32)
    ef = e_ref[...].astype(jnp.float32)
    eexp = dot(ef, rep_ref[...])
    z = (eexp == eidx_ref[...]).astype(jnp.float32)
    intra = dot(z, r_ref[...])
    rowcnt = dot(z, rsum_ref[...])
    rowoff = dot(l128_ref[...], rowcnt)
    counts = jnp.sum(rowcnt, axis=0, keepdims=True)
    pc = jnp.floor((counts + (MG - 1)) * (1.0 / MG)) * MG
    pad_off = dot(pc, l8_ref[...])
    off_exp = dot(rowoff + pad_off, tile8_ref[...])
    pos_sel = dot(z * (intra + off_exp), rept_ref[...])
    pos_ref[...] = pos_sel.astype(jnp.int32)
    cnt_ref[...] = counts.astype(jnp.int32)


@functools.lru_cache(maxsize=1)
def _sc_kernels():
    mesh = plsc.VectorSubcoreMesh(core_axis_name="c", subcore_axis_name="s")

    @functools.partial(
        pl.kernel,
        out_type=jax.ShapeDtypeStruct((RBUF, D), jnp.float32),
        mesh=mesh,
        scratch_types=[
            pltpu.VMEM((CHUNK,), jnp.int32),
            pltpu.VMEM((CHUNK,), jnp.int32),
            pltpu.VMEM((CHUNK, D), jnp.float32),
            pltpu.SemaphoreType.DMA,
        ],
    )
    def _dispatch(tok_hbm, pos_hbm, x_hbm, xs_hbm, tok_v, pos_v, rows_v, sem):
        wid = lax.axis_index("s") * 2 + lax.axis_index("c")
        base = wid * CHUNK
        pltpu.sync_copy(tok_hbm.at[pl.ds(base, CHUNK)], tok_v)
        pltpu.sync_copy(pos_hbm.at[pl.ds(base, CHUNK)], pos_v)
        pltpu.async_copy(x_hbm.at[tok_v], rows_v, sem).wait()
        pltpu.async_copy(rows_v, xs_hbm.at[pos_v], sem).wait()

    @functools.partial(
        pl.kernel,
        out_type=jax.ShapeDtypeStruct((A, D), jnp.float32),
        mesh=mesh,
        scratch_types=[
            pltpu.VMEM((CHUNK,), jnp.int32),
            pltpu.VMEM((CHUNK, D), jnp.float32),
            pltpu.SemaphoreType.DMA,
        ],
    )
    def _collect(pos_hbm, y_hbm, yall_hbm, pos_v, rows_v, sem):
        wid = lax.axis_index("s") * 2 + lax.axis_index("c")
        base = wid * CHUNK
        pltpu.sync_copy(pos_hbm.at[pl.ds(base, CHUNK)], pos_v)
        pltpu.async_copy(y_hbm.at[pos_v], rows_v, sem).wait()
        pltpu.sync_copy(rows_v, yall_hbm.at[pl.ds(base, CHUNK)])

    return _dispatch, _collect


def _grouped_body(ex_ref, xs_ref, wg_ref, wu_ref, wd_ref, y_ref):
    x = xs_ref[...]
    g = jax.nn.silu(jnp.dot(x, wg_ref[0], preferred_element_type=jnp.float32))
    u = jnp.dot(x, wu_ref[0], preferred_element_type=jnp.float32)
    y_ref[...] = jnp.dot(g * u, wd_ref[0], preferred_element_type=jnp.float32)


def _final_body(x_ref, y0_ref, y1_ref, w1_ref, w2_ref, wsg_ref, wsu_ref,
                wsd_ref, out_ref):
    x = x_ref[...]
    g = jax.nn.silu(jnp.dot(x, wsg_ref[...], preferred_element_type=jnp.float32))
    u = jnp.dot(x, wsu_ref[...], preferred_element_type=jnp.float32)
    sh = jnp.dot(g * u, wsd_ref[...], preferred_element_type=jnp.float32)
    out_ref[...] = sh + w1_ref[...] * y0_ref[...] + w2_ref[...] * y1_ref[...]


@jax.jit
def kernel(x, router_w, W_gate, W_up, W_down, Ws_gate, Ws_up, Ws_down):
    e1, e2, w1, w2 = pl.pallas_call(
        _router_body,
        grid=(1,),
        in_specs=[
            pl.BlockSpec((T, D), lambda i: (0, 0)),
            pl.BlockSpec((D, E), lambda i: (0, 0)),
        ],
        out_specs=[
            pl.BlockSpec((T, 1), lambda i: (0, 0)),
            pl.BlockSpec((T, 1), lambda i: (0, 0)),
            pl.BlockSpec((T, 1), lambda i: (0, 0)),
            pl.BlockSpec((T, 1), lambda i: (0, 0)),
        ],
        out_shape=[
            jax.ShapeDtypeStruct((T, 1), jnp.int32),
            jax.ShapeDtypeStruct((T, 1), jnp.int32),
            jax.ShapeDtypeStruct((T, 1), jnp.float32),
            jax.ShapeDtypeStruct((T, 1), jnp.float32),
        ],
    )(x, router_w)

    e_qp = jnp.concatenate([e1[:, 0], e2[:, 0]]).reshape(_Q, _P)
    full = lambda s: pl.BlockSpec(s, lambda i: tuple(0 for _ in s))
    pos_qp, counts = pl.pallas_call(
        _index_body,
        grid=(1,),
        in_specs=[full((_Q, _P)), full((_P, _C)), full((1, _C)),
                  full((_C, _C)), full((_C, E)), full((_Q, _Q)),
                  full((E, E)), full((E, _C)), full((_C, _P))],
        out_specs=[full((_Q, _P)), full((1, E))],
        out_shape=[
            jax.ShapeDtypeStruct((_Q, _P), jnp.int32),
            jax.ShapeDtypeStruct((1, E), jnp.int32),
        ],
    )(e_qp, _REP, _EIDX, _R256, _RSUM, _L128, _L8, _TILE8, _REPT)
    pos_flat = pos_qp.reshape(A)

    pc = ((counts[0] + MG - 1) // MG) * MG
    pad_end = jnp.cumsum(pc)
    tile_start = jnp.arange(NT, dtype=jnp.int32) * MG
    ex_tile = jnp.clip(
        jnp.sum((tile_start[:, None] >= pad_end[None, :]).astype(jnp.int32),
                axis=1), 0, E - 1).astype(jnp.int32)

    _dispatch, _collect = _sc_kernels()
    xs = _dispatch(jnp.asarray(_TOK), pos_flat, x)

    y = pl.pallas_call(
        _grouped_body,
        grid_spec=pltpu.PrefetchScalarGridSpec(
            num_scalar_prefetch=1,
            grid=(NT,),
            in_specs=[
                pl.BlockSpec((MG, D), lambda i, ex: (i, 0)),
                pl.BlockSpec((1, D, F), lambda i, ex: (ex[i], 0, 0)),
                pl.BlockSpec((1, D, F), lambda i, ex: (ex[i], 0, 0)),
                pl.BlockSpec((1, F, D), lambda i, ex: (ex[i], 0, 0)),
            ],
            out_specs=pl.BlockSpec((MG, D), lambda i, ex: (i, 0)),
        ),
        out_shape=jax.ShapeDtypeStruct((RBUF, D), jnp.float32),
        compiler_params=pltpu.CompilerParams(
            dimension_semantics=("arbitrary",),
        ),
    )(ex_tile, xs, W_gate, W_up, W_down)

    yall = _collect(pos_flat, y)
    y0 = yall[:T]
    y1 = yall[T:]

    out = pl.pallas_call(
        _final_body,
        grid=(T // MF,),
        in_specs=[
            pl.BlockSpec((MF, D), lambda i: (i, 0)),
            pl.BlockSpec((MF, D), lambda i: (i, 0)),
            pl.BlockSpec((MF, D), lambda i: (i, 0)),
            pl.BlockSpec((MF, 1), lambda i: (i, 0)),
            pl.BlockSpec((MF, 1), lambda i: (i, 0)),
            pl.BlockSpec((D, F), lambda i: (0, 0)),
            pl.BlockSpec((D, F), lambda i: (0, 0)),
            pl.BlockSpec((F, D), lambda i: (0, 0)),
        ],
        out_specs=pl.BlockSpec((MF, D), lambda i: (i, 0)),
        out_shape=jax.ShapeDtypeStruct((T, D), jnp.float32),
        compiler_params=pltpu.CompilerParams(
            dimension_semantics=("parallel",),
        ),
    )(x, y0, y1, w1, w2, Ws_gate, Ws_up, Ws_down)
    return out


# speedup vs baseline: 2.1497x; 1.0820x over previous
import functools

import numpy as np
import jax
import jax.numpy as jnp
from jax import lax
from jax.experimental import pallas as pl
from jax.experimental.pallas import tpu as pltpu
from jax.experimental.pallas import tpu_sc as plsc

T = 2048
D = 768
F = 1280
E = 8
K = 2
A = T * K
MG = 128
RBUF = A + E * MG
NT = RBUF // MG
NW = 32
CHUNK = A // NW
MF = 512

_P, _Q = 32, 128
_C = _P * E
_cc = np.arange(_C)
_REP = (np.arange(_P)[:, None] == (_cc // E)[None, :]).astype(np.float32)
_EIDX = (_cc % E).astype(np.float32)[None, :]
_R256 = (((_cc % E)[:, None] == (_cc % E)[None, :])
         & ((_cc // E)[:, None] < (_cc // E)[None, :])).astype(np.float32)
_RSUM = ((_cc % E)[:, None] == np.arange(E)[None, :]).astype(np.float32)
_L128 = (np.arange(_Q)[None, :] < np.arange(_Q)[:, None]).astype(np.float32)
_L8 = (np.arange(E)[:, None] < np.arange(E)[None, :]).astype(np.float32)
_TILE8 = (np.arange(E)[:, None] == (_cc % E)[None, :]).astype(np.float32)
_REPT = ((_cc // E)[:, None] == np.arange(_P)[None, :]).astype(np.float32)
_TSTART = (np.arange(NT) * MG).astype(np.float32)[:, None]


def _router_body(x_ref, rw_ref, e1_ref, e2_ref, w1_ref, w2_ref):
    x = x_ref[...]
    logits = jnp.dot(x, rw_ref[...], preferred_element_type=jnp.float32)
    aff = jax.nn.sigmoid(logits)
    ii = jax.lax.broadcasted_iota(jnp.int32, aff.shape, 1)
    m1 = jnp.max(aff, axis=1, keepdims=True)
    e1 = jnp.min(jnp.where(aff == m1, ii, E), axis=1, keepdims=True)
    aff2 = jnp.where(ii == e1, -1.0, aff)
    m2 = jnp.max(aff2, axis=1, keepdims=True)
    e2 = jnp.min(jnp.where(aff2 == m2, ii, E), axis=1, keepdims=True)
    denom = m1 + m2
    e1_ref[...] = e1
    e2_ref[...] = e2
    w1_ref[...] = m1 / denom
    w2_ref[...] = m2 / denom


def _index_body(e_ref, rep_ref, eidx_ref, r_ref, rsum_ref, l128_ref,
                l8_ref, tile8_ref, rept_ref, ts_ref, pos_ref, ex_ref,
                val_ref):
    hi = jax.lax.Precision.HIGHEST
    dot = functools.partial(jnp.dot, precision=hi,
                            preferred_element_type=jnp.float32)
    ef = e_ref[...].astype(jnp.float32)
    eexp = dot(ef, rep_ref[...])
    z = (eexp == eidx_ref[...]).astype(jnp.float32)
    intra = dot(z, r_ref[...])
    rowcnt = dot(z, rsum_ref[...])
    rowoff = dot(l128_ref[...], rowcnt)
    counts = jnp.sum(rowcnt, axis=0, keepdims=True)
    pc = jnp.floor((counts + (MG - 1)) * (1.0 / MG)) * MG
    pad_off = dot(pc, l8_ref[...])
    off_exp = dot(rowoff + pad_off, tile8_ref[...])
    pos_sel = dot(z * (intra + off_exp), rept_ref[...])
    pos_ref[...] = pos_sel.astype(jnp.int32)
    pad_end = pad_off + pc
    ts = ts_ref[...]
    ex = jnp.sum((ts >= pad_end).astype(jnp.float32), axis=1, keepdims=True)
    ex_ref[...] = jnp.clip(ex, 0.0, float(E - 1)).astype(jnp.int32)
    val_ref[...] = (ts < pad_end[:, E - 1:E]).astype(jnp.int32)


@functools.lru_cache(maxsize=1)
def _sc_kernels():
    mesh = plsc.VectorSubcoreMesh(core_axis_name="c", subcore_axis_name="s")

    @functools.partial(
        pl.kernel,
        out_type=jax.ShapeDtypeStruct((RBUF, D), jnp.float32),
        mesh=mesh,
        scratch_types=[
            pltpu.VMEM((CHUNK,), jnp.int32),
            pltpu.VMEM((CHUNK, D), jnp.float32),
            pltpu.SemaphoreType.DMA,
        ],
    )
    def _dispatch(pos_hbm, x_hbm, xs_hbm, pos_v, rows_v, sem):
        wid = lax.axis_index("s") * 2 + lax.axis_index("c")
        base = wid * CHUNK
        tbase = lax.rem(base, T)
        pltpu.sync_copy(pos_hbm.at[pl.ds(base, CHUNK)], pos_v)
        pltpu.sync_copy(x_hbm.at[pl.ds(tbase, CHUNK)], rows_v)
        pltpu.async_copy(rows_v, xs_hbm.at[pos_v], sem).wait()

    @functools.partial(
        pl.kernel,
        out_type=jax.ShapeDtypeStruct((A, D), jnp.float32),
        mesh=mesh,
        scratch_types=[
            pltpu.VMEM((CHUNK,), jnp.int32),
            pltpu.VMEM((CHUNK, D), jnp.float32),
            pltpu.SemaphoreType.DMA,
        ],
    )
    def _collect(pos_hbm, y_hbm, yall_hbm, pos_v, rows_v, sem):
        wid = lax.axis_index("s") * 2 + lax.axis_index("c")
        base = wid * CHUNK
        pltpu.sync_copy(pos_hbm.at[pl.ds(base, CHUNK)], pos_v)
        pltpu.async_copy(y_hbm.at[pos_v], rows_v, sem).wait()
        pltpu.sync_copy(rows_v, yall_hbm.at[pl.ds(base, CHUNK)])

    return _dispatch, _collect


def _grouped_body(ex_ref, val_ref, xs_ref, wg_ref, wu_ref, wd_ref, y_ref):
    i = pl.program_id(0)

    @pl.when(val_ref[i, 0] == 1)
    def _():
        x = xs_ref[...]
        g = jax.nn.silu(jnp.dot(x, wg_ref[0],
                                preferred_element_type=jnp.float32))
        u = jnp.dot(x, wu_ref[0], preferred_element_type=jnp.float32)
        y_ref[...] = jnp.dot(g * u, wd_ref[0],
                             preferred_element_type=jnp.float32)


def _shared_body(x_ref, wsg_ref, wsu_ref, wsd_ref, out_ref):
    x = x_ref[...]
    g = jax.nn.silu(jnp.dot(x, wsg_ref[...], preferred_element_type=jnp.float32))
    u = jnp.dot(x, wsu_ref[...], preferred_element_type=jnp.float32)
    out_ref[...] = jnp.dot(g * u, wsd_ref[...],
                           preferred_element_type=jnp.float32)


def _final_body(sh_ref, y0_ref, y1_ref, w1_ref, w2_ref, out_ref):
    out_ref[...] = (sh_ref[...] + w1_ref[...] * y0_ref[...]
                    + w2_ref[...] * y1_ref[...])


@jax.jit
def kernel(x, router_w, W_gate, W_up, W_down, Ws_gate, Ws_up, Ws_down):
    e1, e2, w1, w2 = pl.pallas_call(
        _router_body,
        grid=(1,),
        in_specs=[
            pl.BlockSpec((T, D), lambda i: (0, 0)),
            pl.BlockSpec((D, E), lambda i: (0, 0)),
        ],
        out_specs=[pl.BlockSpec((T, 1), lambda i: (0, 0))] * 4,
        out_shape=[
            jax.ShapeDtypeStruct((T, 1), jnp.int32),
            jax.ShapeDtypeStruct((T, 1), jnp.int32),
            jax.ShapeDtypeStruct((T, 1), jnp.float32),
            jax.ShapeDtypeStruct((T, 1), jnp.float32),
        ],
    )(x, router_w)

    e_qp = jnp.concatenate([e1[:, 0], e2[:, 0]]).reshape(_Q, _P)
    full = lambda s: pl.BlockSpec(s, lambda i: tuple(0 for _ in s))
    pos_qp, ex_tile, val_tile = pl.pallas_call(
        _index_body,
        grid=(1,),
        in_specs=[full((_Q, _P)), full((_P, _C)), full((1, _C)),
                  full((_C, _C)), full((_C, E)), full((_Q, _Q)),
                  full((E, E)), full((E, _C)), full((_C, _P)),
                  full((NT, 1))],
        out_specs=[full((_Q, _P)), full((NT, 1)), full((NT, 1))],
        out_shape=[
            jax.ShapeDtypeStruct((_Q, _P), jnp.int32),
            jax.ShapeDtypeStruct((NT, 1), jnp.int32),
            jax.ShapeDtypeStruct((NT, 1), jnp.int32),
        ],
    )(e_qp, _REP, _EIDX, _R256, _RSUM, _L128, _L8, _TILE8, _REPT, _TSTART)
    pos_flat = pos_qp.reshape(A)

    sh = pl.pallas_call(
        _shared_body,
        grid=(T // MF,),
        in_specs=[
            pl.BlockSpec((MF, D), lambda i: (i, 0)),
            pl.BlockSpec((D, F), lambda i: (0, 0)),
            pl.BlockSpec((D, F), lambda i: (0, 0)),
            pl.BlockSpec((F, D), lambda i: (0, 0)),
        ],
        out_specs=pl.BlockSpec((MF, D), lambda i: (i, 0)),
        out_shape=jax.ShapeDtypeStruct((T, D), jnp.float32),
        compiler_params=pltpu.CompilerParams(
            dimension_semantics=("parallel",),
        ),
    )(x, Ws_gate, Ws_up, Ws_down)

    _dispatch, _collect = _sc_kernels()
    xs = _dispatch(pos_flat, x)

    y = pl.pallas_call(
        _grouped_body,
        grid_spec=pltpu.PrefetchScalarGridSpec(
            num_scalar_prefetch=2,
            grid=(NT,),
            in_specs=[
                pl.BlockSpec((MG, D), lambda i, ex, vl: (i, 0)),
                pl.BlockSpec((1, D, F), lambda i, ex, vl: (ex[i, 0], 0, 0)),
                pl.BlockSpec((1, D, F), lambda i, ex, vl: (ex[i, 0], 0, 0)),
                pl.BlockSpec((1, F, D), lambda i, ex, vl: (ex[i, 0], 0, 0)),
            ],
            out_specs=pl.BlockSpec((MG, D), lambda i, ex, vl: (i, 0)),
        ),
        out_shape=jax.ShapeDtypeStruct((RBUF, D), jnp.float32),
        compiler_params=pltpu.CompilerParams(
            dimension_semantics=("arbitrary",),
        ),
    )(ex_tile, val_tile, xs, W_gate, W_up, W_down)

    yall = _collect(pos_flat, y)

    nf = T // MF
    out = pl.pallas_call(
        _final_body,
        grid=(nf,),
        in_specs=[
            pl.BlockSpec((MF, D), lambda i: (i, 0)),
            pl.BlockSpec((MF, D), lambda i: (i, 0)),
            pl.BlockSpec((MF, D), lambda i: (i + nf, 0)),
            pl.BlockSpec((MF, 1), lambda i: (i, 0)),
            pl.BlockSpec((MF, 1), lambda i: (i, 0)),
        ],
        out_specs=pl.BlockSpec((MF, D), lambda i: (i, 0)),
        out_shape=jax.ShapeDtypeStruct((T, D), jnp.float32),
        compiler_params=pltpu.CompilerParams(
            dimension_semantics=("parallel",),
        ),
    )(sh, yall, yall, w1, w2)
    return out
